# Initial kernel scaffold; baseline (speedup 1.0000x reference)
import jax
import jax.numpy as jnp
from jax.experimental import pallas as pl


def kernel(x, edge_index, batch, W0a, b0a, W0b, b0b, W1a, b1a, W1b, b1b, W2a, b2a, W2b, b2b, Wc1, bc1, Wc2, bc2):
    raise NotImplementedError("write your pallas kernel here")


# speedup vs baseline: 4.5680x; 4.5680x over previous
import functools

import jax
import jax.numpy as jnp
from jax import lax
from jax.experimental import pallas as pl
from jax.experimental.pallas import tpu as pltpu
from jax.experimental.pallas import tpu_sc as plsc

N = 10000
E = 320000
D = 128
G = 64

NC = 2
NS = 16
EW = E // (NC * NS)
K = 80
NCHUNK = EW // K
UROW = 200
NU = N // UROW
NUIT = -(-NU // NS)


def _segsum_body(h_hbm, src_hbm, dst_hbm, p0_hbm, p1_hbm,
                 acc_sh, src_v, dst_v, rows_v, sem):
    c = lax.axis_index("c")
    s = lax.axis_index("s")

    def init_body(k, carry):
        u = s + k * NS

        @pl.when(u < NU)
        def _():
            r0 = pl.multiple_of(u * UROW, 8)
            pltpu.sync_copy(h_hbm.at[pl.ds(r0, UROW)], acc_sh.at[pl.ds(r0, UROW)])

        return carry

    lax.fori_loop(0, NUIT, init_body, 0)
    plsc.subcore_barrier()

    base = (c * NS + s) * EW

    def body(i, carry):
        off = pl.multiple_of(base + i * K, 8)
        pltpu.sync_copy(src_hbm.at[pl.ds(off, K)], src_v)
        pltpu.sync_copy(dst_hbm.at[pl.ds(off, K)], dst_v)
        pltpu.async_copy(h_hbm.at[src_v], rows_v, sem).wait()
        pltpu.sync_copy(rows_v, acc_sh.at[dst_v], add=True)
        return carry

    lax.fori_loop(0, NCHUNK, body, 0)
    plsc.subcore_barrier()

    def out_body(k, carry):
        u = s + k * NS

        @pl.when(u < NU)
        def _():
            r0 = pl.multiple_of(u * UROW, 8)

            @pl.when(c == 0)
            def _():
                pltpu.sync_copy(acc_sh.at[pl.ds(r0, UROW)],
                                p0_hbm.at[pl.ds(r0, UROW)])

            @pl.when(c == 1)
            def _():
                pltpu.sync_copy(acc_sh.at[pl.ds(r0, UROW)],
                                p1_hbm.at[pl.ds(r0, UROW)])

        return carry

    lax.fori_loop(0, NUIT, out_body, 0)


@functools.lru_cache(maxsize=None)
def _build_segsum():
    return pl.kernel(
        _segsum_body,
        out_type=(jax.ShapeDtypeStruct((N, D), jnp.float32),
                  jax.ShapeDtypeStruct((N, D), jnp.float32)),
        mesh=plsc.VectorSubcoreMesh(core_axis_name="c", subcore_axis_name="s",
                                    num_cores=NC, num_subcores=NS),
        scratch_types=[
            pltpu.VMEM_SHARED((N, D), jnp.float32),
            pltpu.VMEM((K,), jnp.int32),
            pltpu.VMEM((K,), jnp.int32),
            pltpu.VMEM((K, D), jnp.float32),
            pltpu.SemaphoreType.DMA,
        ],
    )


def _segsum(h, src, dst):
    return _build_segsum()(h, src, dst)


BR = 2000


def _mlp_body(p0_ref, p1_ref, h_ref, wa_ref, ba_ref, wb_ref, bb_ref, o_ref):
    z = p0_ref[...] + p1_ref[...] - h_ref[...]
    z = jnp.dot(z, wa_ref[...], preferred_element_type=jnp.float32) + ba_ref[...]
    z = jnp.maximum(z, 0.0)
    z = jnp.dot(z, wb_ref[...], preferred_element_type=jnp.float32) + bb_ref[...]
    o_ref[...] = jnp.maximum(z, 0.0)


def _mlp(p0, p1, h, wa, ba, wb, bb):
    return pl.pallas_call(
        _mlp_body,
        grid=(N // BR,),
        in_specs=[
            pl.BlockSpec((BR, D), lambda i: (i, 0)),
            pl.BlockSpec((BR, D), lambda i: (i, 0)),
            pl.BlockSpec((BR, D), lambda i: (i, 0)),
            pl.BlockSpec((D, D), lambda i: (0, 0)),
            pl.BlockSpec((1, D), lambda i: (0, 0)),
            pl.BlockSpec((D, D), lambda i: (0, 0)),
            pl.BlockSpec((1, D), lambda i: (0, 0)),
        ],
        out_specs=pl.BlockSpec((BR, D), lambda i: (i, 0)),
        out_shape=jax.ShapeDtypeStruct((N, D), jnp.float32),
    )(p0, p1, h, wa, ba, wb, bb)


GPS = 8
NPAD = 10240


def _pool_body(h_ref, bp_ref, wc1_ref, bc1_ref, wc2_ref, bc2_ref, o_ref):
    j = pl.program_id(0)
    bp = bp_ref[...]

    def count_lt(g):
        return jnp.sum((bp < g).astype(jnp.int32))

    pooled_rows = []
    for t in range(GPS):
        g = j * GPS + t
        s0 = count_lt(g)
        s1 = count_lt(g + 1)
        nch = (s1 - s0 + 7) // 8

        def chunk(k, acc, s0=s0, s1=s1):
            base = jnp.minimum(s0 + k * 8, N - 8)
            rows = h_ref[pl.ds(base, 8), :]
            ridx = base + lax.broadcasted_iota(jnp.int32, (8, D), 0)
            m = (ridx >= s0) & (ridx < s1)
            return jnp.maximum(acc, jnp.where(m, rows, -jnp.inf))

        acc = lax.fori_loop(0, nch, chunk, jnp.full((8, D), -jnp.inf))
        pooled_rows.append(jnp.max(acc, axis=0, keepdims=True))

    pooled = jnp.concatenate(pooled_rows, axis=0)
    z = jnp.dot(pooled, wc1_ref[...], preferred_element_type=jnp.float32)
    z = jnp.maximum(z + bc1_ref[...], 0.0)
    y = jnp.dot(z, wc2_ref[...], preferred_element_type=jnp.float32) + bc2_ref[...]
    o_ref[...] = y


def _pool_head(h, bpad, wc1, bc1, wc2, bc2):
    return pl.pallas_call(
        _pool_body,
        grid=(G // GPS,),
        in_specs=[
            pl.BlockSpec((N, D), lambda j: (0, 0)),
            pl.BlockSpec((NPAD // 128, 128), lambda j: (0, 0)),
            pl.BlockSpec((D, D), lambda j: (0, 0)),
            pl.BlockSpec((1, D), lambda j: (0, 0)),
            pl.BlockSpec((D, 1), lambda j: (0, 0)),
            pl.BlockSpec((1, 1), lambda j: (0, 0)),
        ],
        out_specs=pl.BlockSpec((GPS, 1), lambda j: (j, 0)),
        out_shape=jax.ShapeDtypeStruct((G, 1), jnp.float32),
    )(h, bpad, wc1, bc1, wc2, bc2)


def kernel(x, edge_index, batch, W0a, b0a, W0b, b0b, W1a, b1a, W1b, b1b,
           W2a, b2a, W2b, b2b, Wc1, bc1, Wc2, bc2):
    src = edge_index[0]
    dst = edge_index[1]
    h = x
    for wa, ba, wb, bb in ((W0a, b0a, W0b, b0b),
                           (W1a, b1a, W1b, b1b),
                           (W2a, b2a, W2b, b2b)):
        p0, p1 = _segsum(h, src, dst)
        h = _mlp(p0, p1, h, wa, ba.reshape(1, D), wb, bb.reshape(1, D))
    bpad = jnp.full((NPAD,), G, jnp.int32).at[:N].set(batch).reshape(NPAD // 128, 128)
    return _pool_head(h, bpad, Wc1, bc1.reshape(1, D), Wc2, bc2.reshape(1, 1))
